# Initial kernel scaffold; baseline (speedup 1.0000x reference)
import jax
import jax.numpy as jnp
from jax.experimental import pallas as pl


def kernel(x, edge_index, W_mlp, b_mlp, Wq, bq, Wk, bk, Wv, bv, Wo, bo, W_mu, b_mu, VAR, noise):
    raise NotImplementedError("write your pallas kernel here")


# speedup vs baseline: 1.2446x; 1.2446x over previous
import math

import jax
import jax.numpy as jnp
from jax import lax
from jax.experimental import pallas as pl
from jax.experimental.pallas import tpu as pltpu
from jax.experimental.pallas import tpu_sc as plsc

_CONST = (2 * math.pi) ** 0.5
_B, _N, _E, _H, _A, _F, _HEADS = 2, 10000, 160000, 128, 3, 6, 4
_D = _H // _HEADS
_ROW = _B * _A * _H
_FP = 8

_NC, _NS = 2, 16
_NW = _NC * _NS
_RPW = 384
_NPAD = _NW * _RPW
_WP = 3
_WS = _ROW // _WP
_PW = 3200
_NPIECE = _E // _PW
_TRASH = _E
_ECAP = _E + _PW + 16

_MBLK = 1000
_PBLK = 512


def _mlp_body(x_ref, w_ref, b_ref, out_ref):
    for b in range(_B):
        xb = x_ref[b]
        for a in range(_A):
            y = jnp.dot(xb, w_ref[a], preferred_element_type=jnp.float32,
                        precision=lax.Precision.DEFAULT)
            y = jnp.maximum(y + b_ref[a][None, :], 0.0)
            col = b * _A + a
            out_ref[col // 2, :, (col % 2) * _H:(col % 2) * _H + _H] = y


def _mlp(x, W_mlp, b_mlp):
    return pl.pallas_call(
        _mlp_body,
        grid=(_N // _MBLK,),
        in_specs=[
            pl.BlockSpec((_B, _MBLK, _H), lambda i: (0, i, 0)),
            pl.BlockSpec((_A, _H, _H), lambda i: (0, 0, 0)),
            pl.BlockSpec((_A, _H), lambda i: (0, 0)),
        ],
        out_specs=pl.BlockSpec((_WP, _MBLK, _WS), lambda i: (0, i, 0)),
        out_shape=jax.ShapeDtypeStruct((_WP, _N, _WS), jnp.float32),
    )(x, W_mlp, b_mlp)


def _sc_body(xs_hbm, dst_hbm, src_hbm, zeros_hbm, agg_hbm, bins_hbm,
             dstp, srcp, posb, packb, scanb, idx16, rows_v, acc_v):
    c = lax.axis_index("c")
    s = lax.axis_index("s")
    w = s * _NC + c
    lo = w * _RPW
    base = w * _ECAP
    scanb[pl.ds(0, 16)] = jnp.zeros((16,), jnp.int32)
    lanes = lax.iota(jnp.int32, 16)

    def piece1(p, cur):
        pltpu.sync_copy(dst_hbm.at[pl.ds(p * _PW, _PW)], dstp)
        pltpu.sync_copy(src_hbm.at[pl.ds(p * _PW, _PW)], srcp)

        def blk(i, cur):
            d16 = dstp[pl.ds(i * 16, 16)]
            s16 = srcp[pl.ds(i * 16, 16)]
            m = (d16 >= lo) & (d16 < lo + _RPW)
            mi = jnp.where(m, jnp.int32(1), jnp.int32(0))
            scanb[pl.ds(16, 16)] = mi
            for k in (1, 2, 4, 8):
                scanb[pl.ds(16, 16)] = (scanb[pl.ds(16, 16)]
                                        + scanb[pl.ds(16 - k, 16)])
            pf = scanb[pl.ds(16, 16)]
            cnt = pf[15]
            pos = jnp.where(m, cur + pf - 1, _TRASH + i * 16 + lanes)
            packed = jnp.where(m, s16 * 512 + (d16 - lo), 0)
            posb[i // 8, pl.ds((i % 8) * 16, 16)] = base + pos
            packb[i // 8, pl.ds((i % 8) * 16, 16)] = packed
            return cur + cnt

        cur = lax.fori_loop(0, _PW // 16, blk, cur)
        for j in range(_PW // 128):
            pltpu.sync_copy(packb.at[j], bins_hbm.at[posb.at[j]])
        return cur

    cur = lax.fori_loop(0, _NPIECE, piece1, jnp.int32(0))
    plsc.subcore_barrier()

    for t in range(_WP):
        pltpu.sync_copy(zeros_hbm, acc_v.at[pl.ds(0, _RPW)])

        def piece2(p, carry):
            rem = cur - p * _PW
            pltpu.sync_copy(bins_hbm.at[pl.ds(base + p * _PW, _PW)], dstp)
            nb = jnp.clip((rem + 15) // 16, 0, _PW // 16)

            def blk(g, carry2):
                pk = dstp[pl.ds(g * 16, 16)]
                valid = lanes < (rem - g * 16)
                idx16[...] = jnp.clip(jnp.where(valid, (pk >> 9) + t * _N, 0),
                                      0, _WP * _N - 1)
                offs = jnp.minimum(jnp.where(valid, pk & 511, _RPW),
                                   jnp.int32(_RPW))
                pltpu.sync_copy(xs_hbm.at[idx16], rows_v)
                for r in range(16):
                    o = offs[r]
                    for u in range(_WS // 16):
                        sl = pl.ds(u * 16, 16)
                        acc_v[o, sl] = acc_v[o, sl] + rows_v[r, sl]
                return carry2

            lax.fori_loop(0, nb, blk, jnp.int32(0))
            return carry

        npiece = jnp.clip((cur + _PW - 1) // _PW, 0, _NPIECE)
        lax.fori_loop(0, npiece, piece2, jnp.int32(0))
        pltpu.sync_copy(acc_v.at[pl.ds(0, _RPW)],
                        agg_hbm.at[t, pl.ds(lo, _RPW)])


def _sc_agg(xs3_flat, dst, src, zeros):
    f = pl.kernel(
        _sc_body,
        out_type=(
            jax.ShapeDtypeStruct((_WP, _NPAD, _WS), jnp.float32),
            jax.ShapeDtypeStruct((_NW * _ECAP,), jnp.int32),
        ),
        mesh=plsc.VectorSubcoreMesh(core_axis_name="c", subcore_axis_name="s",
                                    num_cores=_NC, num_subcores=_NS),
        scratch_types=[
            pltpu.VMEM((_PW,), jnp.int32),
            pltpu.VMEM((_PW,), jnp.int32),
            pltpu.VMEM((_PW // 128, 128), jnp.int32),
            pltpu.VMEM((_PW // 128, 128), jnp.int32),
            pltpu.VMEM((32,), jnp.int32),
            pltpu.VMEM((16,), jnp.int32),
            pltpu.VMEM((16, _WS), jnp.float32),
            pltpu.VMEM((_RPW + 1, _WS), jnp.float32),
        ],
    )
    agg3, _ = f(xs3_flat, dst, src, zeros)
    return agg3


def _heads_body(agg_ref, wq_ref, bq_ref, wk_ref, bk_ref, wv_ref, bv_ref,
                wo_ref, bo_ref, wmu_ref, bmu_ref, var_ref, noise_ref,
                sa_ref, lp_ref):
    sel = (lax.broadcasted_iota(jnp.int32, (_H, _FP), 0) // _D
           == lax.broadcasted_iota(jnp.int32, (_H, _FP), 1)).astype(jnp.float32)
    expm = (lax.broadcasted_iota(jnp.int32, (_FP, _H), 0)
            == lax.broadcasted_iota(jnp.int32, (_FP, _H), 1) // _D).astype(jnp.float32)
    fmask = lax.broadcasted_iota(jnp.int32, (_PBLK, _FP), 1) < _F
    scale = 1.0 / math.sqrt(_D)

    for b in range(_B):
        q, k, v = [], [], []
        for i in range(_A):
            col = b * _A + i
            g = agg_ref[col // 2, :, (col % 2) * _H:(col % 2) * _H + _H]
            q.append(jnp.dot(g, wq_ref[...], preferred_element_type=jnp.float32,
                             precision=lax.Precision.DEFAULT)
                     + bq_ref[...][None, :])
            k.append(jnp.dot(g, wk_ref[...], preferred_element_type=jnp.float32,
                             precision=lax.Precision.DEFAULT)
                     + bk_ref[...][None, :])
            v.append(jnp.dot(g, wv_ref[...], preferred_element_type=jnp.float32,
                             precision=lax.Precision.DEFAULT)
                     + bv_ref[...][None, :])
        sc = [[jnp.dot(q[i] * k[j], sel, preferred_element_type=jnp.float32,
                     precision=lax.Precision.HIGHEST)
               * scale for j in range(_A)] for i in range(_A)]
        lp_acc = jnp.zeros((_PBLK,), jnp.float32)
        for i in range(_A):
            m = jnp.maximum(jnp.maximum(sc[i][0], sc[i][1]), sc[i][2])
            e = [jnp.exp(sc[i][j] - m) for j in range(_A)]
            den = e[0] + e[1] + e[2]
            o = jnp.zeros((_PBLK, _H), jnp.float32)
            for j in range(_A):
                o = o + jnp.dot(e[j] / den, expm,
                                preferred_element_type=jnp.float32,
                     precision=lax.Precision.HIGHEST) * v[j]
            h = (jnp.dot(o, wo_ref[...], preferred_element_type=jnp.float32,
                         precision=lax.Precision.DEFAULT)
                 + bo_ref[...][None, :])
            mu = (jnp.dot(h, wmu_ref[i], preferred_element_type=jnp.float32,
                          precision=lax.Precision.DEFAULT)
                  + bmu_ref[i][None, :])
            noise_i = noise_ref[b, :, i, :]
            var_i = var_ref[:, i, :]
            sa = mu + noise_i * var_i
            lp = jnp.where(fmask,
                           -jnp.log(_CONST * var_i) - 0.5 * noise_i * noise_i,
                           0.0)
            lp_acc = lp_acc + jnp.sum(lp, axis=1)
            if i == 0:
                tt = jnp.where(fmask, jnp.tanh(sa), -1e30)
                tm = jnp.max(tt, axis=1, keepdims=True)
                et = jnp.exp(tt - tm)
                out = et / jnp.sum(et, axis=1, keepdims=True)
            elif i == 1:
                out = jax.nn.sigmoid(sa)
            else:
                out = jnp.tanh(sa)
            sa_ref[b, :, i, :] = out
        lp_ref[b, :] = lp_acc


def _heads(agg3, Wq, bq, Wk, bk, Wv, bv, Wo, bo, W_mu_p, b_mu_p, VAR_p,
           noise_p):
    full = lambda shape: pl.BlockSpec(shape, lambda i: tuple(0 for _ in shape))
    return pl.pallas_call(
        _heads_body,
        grid=(_NPAD // _PBLK,),
        in_specs=[
            pl.BlockSpec((_WP, _PBLK, _WS), lambda i: (0, i, 0)),
            full((_H, _H)), full((_H,)),
            full((_H, _H)), full((_H,)),
            full((_H, _H)), full((_H,)),
            full((_H, _H)), full((_H,)),
            full((_A, _H, _FP)), full((_A, _FP)),
            pl.BlockSpec((_PBLK, _A, _FP), lambda i: (i, 0, 0)),
            pl.BlockSpec((_B, _PBLK, _A, _FP), lambda i: (0, i, 0, 0)),
        ],
        out_specs=[
            pl.BlockSpec((_B, _PBLK, _A, _FP), lambda i: (0, i, 0, 0)),
            pl.BlockSpec((_B, _PBLK), lambda i: (0, i)),
        ],
        out_shape=[
            jax.ShapeDtypeStruct((_B, _NPAD, _A, _FP), jnp.float32),
            jax.ShapeDtypeStruct((_B, _NPAD), jnp.float32),
        ],
    )(agg3, Wq, bq, Wk, bk, Wv, bv, Wo, bo, W_mu_p, b_mu_p, VAR_p, noise_p)


def kernel(x, edge_index, W_mlp, b_mlp, Wq, bq, Wk, bk, Wv, bv, Wo, bo,
           W_mu, b_mu, VAR, noise):
    xs3 = _mlp(x, W_mlp, b_mlp)
    xs3_flat = xs3.reshape(_WP * _N, _WS)

    dst = edge_index[0]
    src = edge_index[1]
    zeros = jnp.zeros((_RPW, _WS), jnp.float32)
    agg3 = _sc_agg(xs3_flat, dst, src, zeros)

    W_mu_p = jnp.pad(W_mu, ((0, 0), (0, 0), (0, _FP - _F)))
    b_mu_p = jnp.pad(b_mu, ((0, 0), (0, _FP - _F)))
    VAR_p = jnp.pad(VAR, ((0, _NPAD - _N), (0, 0), (0, _FP - _F)),
                    constant_values=1.0)
    noise_p = jnp.pad(noise, ((0, 0), (0, _NPAD - _N), (0, 0), (0, _FP - _F)))

    sa_p, lp_p = _heads(agg3, Wq, bq, Wk, bk, Wv, bv, Wo, bo,
                        W_mu_p, b_mu_p, VAR_p, noise_p)
    return sa_p[:, :_N, :, :_F], lp_p[:, :_N]


# speedup vs baseline: 1.3313x; 1.0697x over previous
import math

import jax
import jax.numpy as jnp
from jax import lax
from jax.experimental import pallas as pl
from jax.experimental.pallas import tpu as pltpu
from jax.experimental.pallas import tpu_sc as plsc

_CONST = (2 * math.pi) ** 0.5
_B, _N, _E, _H, _A, _F, _HEADS = 2, 10000, 160000, 128, 3, 6, 4
_D = _H // _HEADS
_ROW = _B * _A * _H
_FP = 8

_NC, _NS = 2, 16
_NW = _NC * _NS
_RPW = 320
_NPAD = _NW * _RPW
_WP = 3
_WS = _ROW // _WP
_PW = 3200
_NPIECE = _E // _PW
_TRASH = _E
_ECAP = _E + _PW + 16

_MBLK = 1000
_PBLK = 512


def _mlp_body(x_ref, w_ref, b_ref, out_ref):
    for b in range(_B):
        xb = x_ref[b]
        for a in range(_A):
            y = jnp.dot(xb, w_ref[a], preferred_element_type=jnp.float32,
                        precision=lax.Precision.DEFAULT)
            y = jnp.maximum(y + b_ref[a][None, :], 0.0)
            col = b * _A + a
            out_ref[col // 2, :, (col % 2) * _H:(col % 2) * _H + _H] = y


def _mlp(x, W_mlp, b_mlp):
    return pl.pallas_call(
        _mlp_body,
        grid=(_N // _MBLK,),
        in_specs=[
            pl.BlockSpec((_B, _MBLK, _H), lambda i: (0, i, 0)),
            pl.BlockSpec((_A, _H, _H), lambda i: (0, 0, 0)),
            pl.BlockSpec((_A, _H), lambda i: (0, 0)),
        ],
        out_specs=pl.BlockSpec((_WP, _MBLK, _WS), lambda i: (0, i, 0)),
        out_shape=jax.ShapeDtypeStruct((_WP, _N, _WS), jnp.float32),
    )(x, W_mlp, b_mlp)


def _sc_body(xs_hbm, dst_hbm, src_hbm, zeros_hbm, agg_hbm, bins_hbm,
             dstp, srcp, posb, packb, scanb, idx16, rows_v, acc_v):
    c = lax.axis_index("c")
    s = lax.axis_index("s")
    w = s * _NC + c
    lo = w * _RPW
    base = w * _ECAP
    scanb[pl.ds(0, 16)] = jnp.zeros((16,), jnp.int32)
    lanes = lax.iota(jnp.int32, 16)

    def piece1(p, cur):
        pltpu.sync_copy(dst_hbm.at[pl.ds(p * _PW, _PW)], dstp)
        pltpu.sync_copy(src_hbm.at[pl.ds(p * _PW, _PW)], srcp)

        def blk(i, cur):
            d16 = dstp[pl.ds(i * 16, 16)]
            s16 = srcp[pl.ds(i * 16, 16)]
            m = (d16 >= lo) & (d16 < lo + _RPW)
            mi = jnp.where(m, jnp.int32(1), jnp.int32(0))
            scanb[pl.ds(16, 16)] = mi
            for k in (1, 2, 4, 8):
                scanb[pl.ds(16, 16)] = (scanb[pl.ds(16, 16)]
                                        + scanb[pl.ds(16 - k, 16)])
            pf = scanb[pl.ds(16, 16)]
            cnt = pf[15]
            pos = jnp.where(m, cur + pf - 1, _TRASH + i * 16 + lanes)
            packed = jnp.where(m, s16 * 512 + (d16 - lo), 0)
            posb[i // 8, pl.ds((i % 8) * 16, 16)] = base + pos
            packb[i // 8, pl.ds((i % 8) * 16, 16)] = packed
            return cur + cnt

        cur = lax.fori_loop(0, _PW // 16, blk, cur)
        for j in range(_PW // 128):
            pltpu.sync_copy(packb.at[j], bins_hbm.at[posb.at[j]])
        return cur

    cur = lax.fori_loop(0, _NPIECE, piece1, jnp.int32(0))
    plsc.subcore_barrier()

    for t in range(_WP):
        pltpu.sync_copy(zeros_hbm, acc_v.at[pl.ds(0, _RPW)])

        def piece2(p, carry):
            rem = cur - p * _PW
            pltpu.sync_copy(bins_hbm.at[pl.ds(base + p * _PW, _PW)], dstp)
            nb = jnp.clip((rem + 31) // 32, 0, _PW // 32)

            def blk(g, carry2):
                offs2 = []
                for hh in range(2):
                    pk = dstp[pl.ds(g * 32 + hh * 16, 16)]
                    valid = lanes < (rem - g * 32 - hh * 16)
                    idx16[pl.ds(hh * 16, 16)] = jnp.clip(
                        jnp.where(valid, (pk >> 9) + t * _N, 0),
                        0, _WP * _N - 1)
                    offs2.append(jnp.minimum(
                        jnp.where(valid, pk & 511, _RPW), jnp.int32(_RPW)))
                pltpu.sync_copy(xs_hbm.at[idx16], rows_v)
                for hh in range(2):
                    for r in range(16):
                        o = offs2[hh][r]
                        for u in range(_WS // 16):
                            sl = pl.ds(u * 16, 16)
                            acc_v[o, sl] = (acc_v[o, sl]
                                            + rows_v[hh * 16 + r, sl])
                return carry2

            lax.fori_loop(0, nb, blk, jnp.int32(0))
            return carry

        npiece = jnp.clip((cur + _PW - 1) // _PW, 0, _NPIECE)
        lax.fori_loop(0, npiece, piece2, jnp.int32(0))
        pltpu.sync_copy(acc_v.at[pl.ds(0, _RPW)],
                        agg_hbm.at[t, pl.ds(lo, _RPW)])


def _sc_agg(xs3_flat, dst, src, zeros):
    f = pl.kernel(
        _sc_body,
        out_type=(
            jax.ShapeDtypeStruct((_WP, _NPAD, _WS), jnp.float32),
            jax.ShapeDtypeStruct((_NW * _ECAP,), jnp.int32),
        ),
        mesh=plsc.VectorSubcoreMesh(core_axis_name="c", subcore_axis_name="s",
                                    num_cores=_NC, num_subcores=_NS),
        scratch_types=[
            pltpu.VMEM((_PW,), jnp.int32),
            pltpu.VMEM((_PW,), jnp.int32),
            pltpu.VMEM((_PW // 128, 128), jnp.int32),
            pltpu.VMEM((_PW // 128, 128), jnp.int32),
            pltpu.VMEM((32,), jnp.int32),
            pltpu.VMEM((32,), jnp.int32),
            pltpu.VMEM((32, _WS), jnp.float32),
            pltpu.VMEM((_RPW + 1, _WS), jnp.float32),
        ],
    )
    agg3, _ = f(xs3_flat, dst, src, zeros)
    return agg3


def _heads_body(agg_ref, wq_ref, bq_ref, wk_ref, bk_ref, wv_ref, bv_ref,
                wo_ref, bo_ref, wmu_ref, bmu_ref, var_ref, noise_ref,
                sa_ref, lp_ref):
    sel = (lax.broadcasted_iota(jnp.int32, (_H, _FP), 0) // _D
           == lax.broadcasted_iota(jnp.int32, (_H, _FP), 1)).astype(jnp.float32)
    expm = (lax.broadcasted_iota(jnp.int32, (_FP, _H), 0)
            == lax.broadcasted_iota(jnp.int32, (_FP, _H), 1) // _D).astype(jnp.float32)
    fmask = lax.broadcasted_iota(jnp.int32, (_PBLK, _FP), 1) < _F
    scale = 1.0 / math.sqrt(_D)

    for b in range(_B):
        q, k, v = [], [], []
        for i in range(_A):
            col = b * _A + i
            g = agg_ref[col // 2, :, (col % 2) * _H:(col % 2) * _H + _H]
            q.append(jnp.dot(g, wq_ref[...], preferred_element_type=jnp.float32,
                             precision=lax.Precision.DEFAULT)
                     + bq_ref[...][None, :])
            k.append(jnp.dot(g, wk_ref[...], preferred_element_type=jnp.float32,
                             precision=lax.Precision.DEFAULT)
                     + bk_ref[...][None, :])
            v.append(jnp.dot(g, wv_ref[...], preferred_element_type=jnp.float32,
                             precision=lax.Precision.DEFAULT)
                     + bv_ref[...][None, :])
        sc = [[jnp.dot(q[i] * k[j], sel, preferred_element_type=jnp.float32,
                     precision=lax.Precision.HIGHEST)
               * scale for j in range(_A)] for i in range(_A)]
        lp_acc = jnp.zeros((_PBLK,), jnp.float32)
        for i in range(_A):
            m = jnp.maximum(jnp.maximum(sc[i][0], sc[i][1]), sc[i][2])
            e = [jnp.exp(sc[i][j] - m) for j in range(_A)]
            den = e[0] + e[1] + e[2]
            o = jnp.zeros((_PBLK, _H), jnp.float32)
            for j in range(_A):
                o = o + jnp.dot(e[j] / den, expm,
                                preferred_element_type=jnp.float32,
                     precision=lax.Precision.HIGHEST) * v[j]
            h = (jnp.dot(o, wo_ref[...], preferred_element_type=jnp.float32,
                         precision=lax.Precision.DEFAULT)
                 + bo_ref[...][None, :])
            mu = (jnp.dot(h, wmu_ref[i], preferred_element_type=jnp.float32,
                          precision=lax.Precision.DEFAULT)
                  + bmu_ref[i][None, :])
            noise_i = noise_ref[b, :, i, :]
            var_i = var_ref[:, i, :]
            sa = mu + noise_i * var_i
            lp = jnp.where(fmask,
                           -jnp.log(_CONST * var_i) - 0.5 * noise_i * noise_i,
                           0.0)
            lp_acc = lp_acc + jnp.sum(lp, axis=1)
            if i == 0:
                tt = jnp.where(fmask, jnp.tanh(sa), -1e30)
                tm = jnp.max(tt, axis=1, keepdims=True)
                et = jnp.exp(tt - tm)
                out = et / jnp.sum(et, axis=1, keepdims=True)
            elif i == 1:
                out = jax.nn.sigmoid(sa)
            else:
                out = jnp.tanh(sa)
            sa_ref[b, :, i, :] = out
        lp_ref[b, :] = lp_acc


def _heads(agg3, Wq, bq, Wk, bk, Wv, bv, Wo, bo, W_mu_p, b_mu_p, VAR_p,
           noise_p):
    full = lambda shape: pl.BlockSpec(shape, lambda i: tuple(0 for _ in shape))
    return pl.pallas_call(
        _heads_body,
        grid=(_NPAD // _PBLK,),
        in_specs=[
            pl.BlockSpec((_WP, _PBLK, _WS), lambda i: (0, i, 0)),
            full((_H, _H)), full((_H,)),
            full((_H, _H)), full((_H,)),
            full((_H, _H)), full((_H,)),
            full((_H, _H)), full((_H,)),
            full((_A, _H, _FP)), full((_A, _FP)),
            pl.BlockSpec((_PBLK, _A, _FP), lambda i: (i, 0, 0)),
            pl.BlockSpec((_B, _PBLK, _A, _FP), lambda i: (0, i, 0, 0)),
        ],
        out_specs=[
            pl.BlockSpec((_B, _PBLK, _A, _FP), lambda i: (0, i, 0, 0)),
            pl.BlockSpec((_B, _PBLK), lambda i: (0, i)),
        ],
        out_shape=[
            jax.ShapeDtypeStruct((_B, _NPAD, _A, _FP), jnp.float32),
            jax.ShapeDtypeStruct((_B, _NPAD), jnp.float32),
        ],
    )(agg3, Wq, bq, Wk, bk, Wv, bv, Wo, bo, W_mu_p, b_mu_p, VAR_p, noise_p)


def kernel(x, edge_index, W_mlp, b_mlp, Wq, bq, Wk, bk, Wv, bv, Wo, bo,
           W_mu, b_mu, VAR, noise):
    xs3 = _mlp(x, W_mlp, b_mlp)
    xs3_flat = xs3.reshape(_WP * _N, _WS)

    dst = edge_index[0]
    src = edge_index[1]
    zeros = jnp.zeros((_RPW, _WS), jnp.float32)
    agg3 = _sc_agg(xs3_flat, dst, src, zeros)

    W_mu_p = jnp.pad(W_mu, ((0, 0), (0, 0), (0, _FP - _F)))
    b_mu_p = jnp.pad(b_mu, ((0, 0), (0, _FP - _F)))
    VAR_p = jnp.pad(VAR, ((0, _NPAD - _N), (0, 0), (0, _FP - _F)),
                    constant_values=1.0)
    noise_p = jnp.pad(noise, ((0, 0), (0, _NPAD - _N), (0, 0), (0, _FP - _F)))

    sa_p, lp_p = _heads(agg3, Wq, bq, Wk, bk, Wv, bv, Wo, bo,
                        W_mu_p, b_mu_p, VAR_p, noise_p)
    return sa_p[:, :_N, :, :_F], lp_p[:, :_N]
